# baseline (speedup 1.0000x reference)
import functools

import numpy as np
import jax
import jax.numpy as jnp
from jax import lax
from jax.experimental import pallas as pl
from jax.experimental.pallas import tpu as pltpu
from jax.experimental.pallas import tpu_sc as plsc

F = 128
FS = 32
NSHARD = 4
NTILES = 16
CCHUNK = 80


def _mlp_tc(q2, w1t, b1, w2pt, b2p):
    n = q2.shape[0]
    blk = 400

    def body(q_ref, w1t_ref, b1_ref, w2pt_ref, b2p_ref, out_ref):
        h = jnp.dot(q_ref[...], w1t_ref[...], preferred_element_type=jnp.float32)
        h = h + b1_ref[...]
        h = h * jax.nn.sigmoid(h)
        x = jnp.dot(h, w2pt_ref[...], preferred_element_type=jnp.float32)
        out_ref[...] = x + b2p_ref[...]

    return pl.pallas_call(
        body,
        grid=(n // blk,),
        in_specs=[
            pl.BlockSpec((blk, F), lambda i: (i, 0)),
            pl.BlockSpec((F, F), lambda i: (0, 0)),
            pl.BlockSpec((1, F), lambda i: (0, 0)),
            pl.BlockSpec((F, 3 * F), lambda i: (0, 0)),
            pl.BlockSpec((1, 3 * F), lambda i: (0, 0)),
        ],
        out_specs=pl.BlockSpec((blk, 3 * F), lambda i: (i, 0)),
        out_shape=jax.ShapeDtypeStruct((n, 3 * F), jnp.float32),
    )(q2, w1t, b1.reshape(1, F), w2pt, b2p.reshape(1, 3 * F))


def _edges_sc(xmu4, init4, wij, dirij, idx_i, idx_j):
    n = init4.shape[1]
    e = wij.shape[0]
    e_tile = e // NTILES
    n_tile = n // NTILES
    nchunk = e_tile // CCHUNK
    npair = (nchunk - 1) // 2
    mesh = plsc.VectorSubcoreMesh(core_axis_name="c", subcore_axis_name="s")
    vm = pltpu.VMEM

    @functools.partial(
        pl.kernel,
        mesh=mesh,
        compiler_params=pltpu.CompilerParams(use_tc_tiling_on_sc=False),
        out_type=[
            jax.ShapeDtypeStruct((n, 1, F), jnp.float32),
            jax.ShapeDtypeStruct((n, 3, F), jnp.float32),
        ],
        scratch_types=[
            pltpu.VMEM_SHARED((n, F), jnp.float32),
            [vm((CCHUNK,), jnp.int32)] * 2,
            [vm((CCHUNK,), jnp.int32)] * 2,
            [vm((CCHUNK * 3 + 16,), jnp.float32)] * 2,
            vm((CCHUNK, FS), jnp.float32),
            vm((CCHUNK, FS), jnp.float32),
            vm((CCHUNK, FS), jnp.float32),
            [vm((CCHUNK, 6 * FS), jnp.float32)] * 2,
            vm((CCHUNK, 4 * FS), jnp.float32),
            [pltpu.SemaphoreType.DMA] * 2,
            [pltpu.SemaphoreType.DMA] * 2,
            pltpu.SemaphoreType.DMA,
        ],
    )
    def k(xmu4_h, init4_h, wij_h, dir_h, idxi_h, idxj_h, qout_h, muout_h,
          acc, idxi_v, idxj_v, dir_v, wd_v, wa_v, wb_v, xm_v, pay_v,
          semj, semlg, semw):
        c = lax.axis_index("c")
        s = lax.axis_index("s")
        rows0 = s * n_tile
        ebase = s * e_tile

        def issue_idx(kk, b):
            e0 = ebase + kk * CCHUNK
            pltpu.async_copy(idxj_h.at[pl.ds(e0, CCHUNK)], idxj_v[b], semj[b])
            pltpu.async_copy(idxi_h.at[pl.ds(e0, CCHUNK)], idxi_v[b], semlg[b])
            pltpu.async_copy(dir_h.at[pl.ds(3 * e0, 3 * CCHUNK)],
                             dir_v[b].at[pl.ds(0, 3 * CCHUNK)], semlg[b])

        def issue_wij(kk, shard):
            e0 = ebase + kk * CCHUNK
            for col, dst in ((0, wd_v), (F, wa_v), (2 * F, wb_v)):
                pltpu.async_copy(
                    wij_h.at[pl.ds(e0, CCHUNK), pl.ds(col + shard * FS, FS)],
                    dst, semw)

        def wait_j(b):
            pltpu.make_async_copy(idxj_h.at[pl.ds(0, CCHUNK)], idxj_v[b],
                                  semj[b]).wait()

        def wait_wij():
            for dst in (wd_v, wa_v, wb_v):
                pltpu.make_async_copy(
                    wij_h.at[pl.ds(0, CCHUNK), pl.ds(0, FS)], dst, semw).wait()

        def issue_gathers(b, shard):
            pltpu.async_copy(xmu4_h.at[shard].at[idxj_v[b]], xm_v[b], semlg[b])

        def wait_lg(b):
            pltpu.make_async_copy(idxi_h.at[pl.ds(0, CCHUNK)], idxi_v[b],
                                  semlg[b]).wait()
            pltpu.make_async_copy(dir_h.at[pl.ds(0, 3 * CCHUNK)],
                                  dir_v[b].at[pl.ds(0, 3 * CCHUNK)],
                                  semlg[b]).wait()
            pltpu.make_async_copy(xmu4_h.at[0].at[idxj_v[b]], xm_v[b],
                                  semlg[b]).wait()

        def compute_scatter(b):
            def edge_body(ei, _):
                dvec = dir_v[b][pl.ds(3 * ei, 16)]
                d0 = dvec[0]
                d1 = dvec[1]
                d2 = dvec[2]
                for h in range(FS // 16):
                    hs = 16 * h
                    pay_v[ei, pl.ds(hs, 16)] = (
                        wd_v[ei, pl.ds(hs, 16)] * xm_v[b][ei, pl.ds(hs, 16)])
                    a = wa_v[ei, pl.ds(hs, 16)] * xm_v[b][ei, pl.ds(FS + hs, 16)]
                    bb = (wb_v[ei, pl.ds(hs, 16)]
                          * xm_v[b][ei, pl.ds(2 * FS + hs, 16)])
                    for d, dd in enumerate((d0, d1, d2)):
                        pay_v[ei, pl.ds(FS + FS * d + hs, 16)] = (
                            a * dd
                            + bb * xm_v[b][ei, pl.ds(3 * FS + FS * d + hs, 16)])
                return 0

            lax.fori_loop(0, CCHUNK, edge_body, 0, unroll=8)
            pltpu.sync_copy(pay_v, acc.at[idxi_v[b]], add=True)

        for p in range(2):
            shard = 2 * p + c

            pltpu.sync_copy(init4_h.at[shard, pl.ds(rows0, n_tile)],
                            acc.at[pl.ds(rows0, n_tile)])
            plsc.subcore_barrier()

            issue_idx(0, 0)
            issue_idx(1, 1)
            issue_wij(0, shard)
            wait_j(0)
            issue_gathers(0, shard)

            def pair_body(kp, _):
                for b in (0, 1):
                    kk = 2 * kp + b
                    wait_j(1 - b)
                    issue_gathers(1 - b, shard)
                    wait_wij()
                    wait_lg(b)
                    compute_scatter(b)
                    issue_wij(kk + 1, shard)

                    @pl.when(kk + 2 < nchunk)
                    def _():
                        issue_idx(kk + 2, b)
                return 0

            lax.fori_loop(0, npair, pair_body, 0)
            wait_wij()
            wait_lg(0)
            compute_scatter(0)
            plsc.subcore_barrier()

            col = shard * FS
            pltpu.sync_copy(acc.at[pl.ds(rows0, n_tile), pl.ds(0, FS)],
                            qout_h.at[pl.ds(rows0, n_tile), 0, pl.ds(col, FS)])
            for d in range(3):
                pltpu.sync_copy(
                    acc.at[pl.ds(rows0, n_tile), pl.ds(FS + FS * d, FS)],
                    muout_h.at[pl.ds(rows0, n_tile), d, pl.ds(col, FS)])
            if p == 0:
                plsc.subcore_barrier()

    return k(xmu4, init4, wij, dirij, idx_i, idx_j)


def kernel(q, mu, Wij, dir_ij, idx_i, idx_j, n_atoms, W1, b1, W2, b2):
    n = q.shape[0]
    e = Wij.shape[0]
    idx_i = idx_i.astype(jnp.int32)
    idx_j = idx_j.astype(jnp.int32)

    perm = np.concatenate([
        np.concatenate([np.arange(FS * s, FS * s + FS),
                        np.arange(F + FS * s, F + FS * s + FS),
                        np.arange(2 * F + FS * s, 2 * F + FS * s + FS)])
        for s in range(NSHARD)
    ])
    w2p = W2[perm]
    b2p = b2[perm]

    xp = _mlp_tc(q[:, 0, :], W1.T, b1, w2p.T, b2p)
    x4 = xp.reshape(n, NSHARD, 3 * FS)

    mu_r = mu.reshape(n, 3, NSHARD, FS).transpose(0, 2, 1, 3)
    mu4 = mu_r.reshape(n, NSHARD, 3 * FS)
    xmu4 = jnp.concatenate([x4, mu4], axis=-1).transpose(1, 0, 2)
    q_r = q.reshape(n, 1, NSHARD, FS).transpose(0, 2, 1, 3)
    init4 = jnp.concatenate([q_r, mu_r], axis=2)
    init4 = init4.transpose(1, 0, 2, 3).reshape(NSHARD, n, 4 * FS)

    q_out, mu_out = _edges_sc(xmu4, init4, Wij.reshape(e, 3 * F),
                              dir_ij.reshape(-1), idx_i, idx_j)
    return (q_out.astype(q.dtype), mu_out.astype(mu.dtype))


# speedup vs baseline: 1.7779x; 1.7779x over previous
import functools

import numpy as np
import jax
import jax.numpy as jnp
from jax import lax
from jax.experimental import pallas as pl
from jax.experimental.pallas import tpu as pltpu
from jax.experimental.pallas import tpu_sc as plsc

F = 128
FS = 32
NSHARD = 4
NTILES = 16
CCHUNK = 80


def _mlp_tc(q2, w1t, b1, w2pt, b2p):
    n = q2.shape[0]
    blk = 400

    def body(q_ref, w1t_ref, b1_ref, w2pt_ref, b2p_ref, out_ref):
        h = jnp.dot(q_ref[...], w1t_ref[...], preferred_element_type=jnp.float32)
        h = h + b1_ref[...]
        h = h * jax.nn.sigmoid(h)
        x = jnp.dot(h, w2pt_ref[...], preferred_element_type=jnp.float32)
        out_ref[...] = x + b2p_ref[...]

    return pl.pallas_call(
        body,
        grid=(n // blk,),
        in_specs=[
            pl.BlockSpec((blk, F), lambda i: (i, 0)),
            pl.BlockSpec((F, F), lambda i: (0, 0)),
            pl.BlockSpec((1, F), lambda i: (0, 0)),
            pl.BlockSpec((F, 3 * F), lambda i: (0, 0)),
            pl.BlockSpec((1, 3 * F), lambda i: (0, 0)),
        ],
        out_specs=pl.BlockSpec((blk, 3 * F), lambda i: (i, 0)),
        out_shape=jax.ShapeDtypeStruct((n, 3 * F), jnp.float32),
    )(q2, w1t, b1.reshape(1, F), w2pt, b2p.reshape(1, 3 * F))


def _edges_sc(xmu4, init4, wij, dirij, idx_i, idx_j):
    n = init4.shape[1]
    e = wij.shape[0]
    e_tile = e // NTILES
    n_tile = n // NTILES
    nchunk = e_tile // CCHUNK
    npair = (nchunk - 1) // 2
    mesh = plsc.VectorSubcoreMesh(core_axis_name="c", subcore_axis_name="s")
    vm = pltpu.VMEM

    @functools.partial(
        pl.kernel,
        mesh=mesh,
        compiler_params=pltpu.CompilerParams(use_tc_tiling_on_sc=False),
        out_type=[
            jax.ShapeDtypeStruct((n, 1, F), jnp.float32),
            jax.ShapeDtypeStruct((n, 3, F), jnp.float32),
        ],
        scratch_types=[
            pltpu.VMEM_SHARED((n, F), jnp.float32),
            [vm((CCHUNK,), jnp.int32)] * 2,
            [vm((CCHUNK,), jnp.int32)] * 2,
            [vm((CCHUNK * 3 + 16,), jnp.float32)] * 2,
            vm((CCHUNK, FS), jnp.float32),
            vm((CCHUNK, FS), jnp.float32),
            vm((CCHUNK, FS), jnp.float32),
            [vm((CCHUNK, 6 * FS), jnp.float32)] * 2,
            vm((CCHUNK, 4 * FS), jnp.float32),
            [pltpu.SemaphoreType.DMA] * 2,
            [pltpu.SemaphoreType.DMA] * 2,
            pltpu.SemaphoreType.DMA,
        ],
    )
    def k(xmu4_h, init4_h, wij_h, dir_h, idxi_h, idxj_h, qout_h, muout_h,
          acc, idxi_v, idxj_v, dir_v, wd_v, wa_v, wb_v, xm_v, pay_v,
          semj, semlg, semw):
        c = lax.axis_index("c")
        s = lax.axis_index("s")
        rows0 = s * n_tile
        ebase = s * e_tile

        def issue_idx(kk, b):
            e0 = ebase + kk * CCHUNK
            pltpu.async_copy(idxj_h.at[pl.ds(e0, CCHUNK)], idxj_v[b], semj[b])
            pltpu.async_copy(idxi_h.at[pl.ds(e0, CCHUNK)], idxi_v[b], semlg[b])
            pltpu.async_copy(dir_h.at[pl.ds(3 * e0, 3 * CCHUNK)],
                             dir_v[b].at[pl.ds(0, 3 * CCHUNK)], semlg[b])

        def issue_wij(kk, shard):
            e0 = ebase + kk * CCHUNK
            for col, dst in ((0, wd_v), (F, wa_v), (2 * F, wb_v)):
                pltpu.async_copy(
                    wij_h.at[pl.ds(e0, CCHUNK), pl.ds(col + shard * FS, FS)],
                    dst, semw)

        def wait_j(b):
            pltpu.make_async_copy(idxj_h.at[pl.ds(0, CCHUNK)], idxj_v[b],
                                  semj[b]).wait()

        def wait_wij():
            for dst in (wd_v, wa_v, wb_v):
                pltpu.make_async_copy(
                    wij_h.at[pl.ds(0, CCHUNK), pl.ds(0, FS)], dst, semw).wait()

        def issue_gathers(b, shard):
            pltpu.async_copy(xmu4_h.at[shard].at[idxj_v[b]], xm_v[b], semlg[b])

        def wait_lg(b):
            pltpu.make_async_copy(idxi_h.at[pl.ds(0, CCHUNK)], idxi_v[b],
                                  semlg[b]).wait()
            pltpu.make_async_copy(dir_h.at[pl.ds(0, 3 * CCHUNK)],
                                  dir_v[b].at[pl.ds(0, 3 * CCHUNK)],
                                  semlg[b]).wait()
            pltpu.make_async_copy(xmu4_h.at[0].at[idxj_v[b]], xm_v[b],
                                  semlg[b]).wait()

        def compute_scatter(b):
            @plsc.parallel_loop(0, CCHUNK, unroll=4)
            def edge_body(ei):
                dvec = dir_v[b][pl.ds(3 * ei, 16)]
                d0 = dvec[0]
                d1 = dvec[1]
                d2 = dvec[2]
                for h in range(FS // 16):
                    hs = 16 * h
                    pay_v[ei, pl.ds(hs, 16)] = (
                        wd_v[ei, pl.ds(hs, 16)] * xm_v[b][ei, pl.ds(hs, 16)])
                    a = wa_v[ei, pl.ds(hs, 16)] * xm_v[b][ei, pl.ds(FS + hs, 16)]
                    bb = (wb_v[ei, pl.ds(hs, 16)]
                          * xm_v[b][ei, pl.ds(2 * FS + hs, 16)])
                    for d, dd in enumerate((d0, d1, d2)):
                        pay_v[ei, pl.ds(FS + FS * d + hs, 16)] = (
                            a * dd
                            + bb * xm_v[b][ei, pl.ds(3 * FS + FS * d + hs, 16)])
            pltpu.sync_copy(pay_v, acc.at[idxi_v[b]], add=True)

        for p in range(2):
            shard = 2 * p + c

            pltpu.sync_copy(init4_h.at[shard, pl.ds(rows0, n_tile)],
                            acc.at[pl.ds(rows0, n_tile)])
            plsc.subcore_barrier()

            issue_idx(0, 0)
            issue_idx(1, 1)
            issue_wij(0, shard)
            wait_j(0)
            issue_gathers(0, shard)

            def pair_body(kp, _):
                for b in (0, 1):
                    kk = 2 * kp + b
                    wait_j(1 - b)
                    issue_gathers(1 - b, shard)
                    wait_wij()
                    wait_lg(b)
                    compute_scatter(b)
                    issue_wij(kk + 1, shard)

                    @pl.when(kk + 2 < nchunk)
                    def _():
                        issue_idx(kk + 2, b)
                return 0

            lax.fori_loop(0, npair, pair_body, 0)
            wait_wij()
            wait_lg(0)
            compute_scatter(0)
            plsc.subcore_barrier()

            col = shard * FS
            pltpu.sync_copy(acc.at[pl.ds(rows0, n_tile), pl.ds(0, FS)],
                            qout_h.at[pl.ds(rows0, n_tile), 0, pl.ds(col, FS)])
            for d in range(3):
                pltpu.sync_copy(
                    acc.at[pl.ds(rows0, n_tile), pl.ds(FS + FS * d, FS)],
                    muout_h.at[pl.ds(rows0, n_tile), d, pl.ds(col, FS)])
            if p == 0:
                plsc.subcore_barrier()

    return k(xmu4, init4, wij, dirij, idx_i, idx_j)


def kernel(q, mu, Wij, dir_ij, idx_i, idx_j, n_atoms, W1, b1, W2, b2):
    n = q.shape[0]
    e = Wij.shape[0]
    idx_i = idx_i.astype(jnp.int32)
    idx_j = idx_j.astype(jnp.int32)

    perm = np.concatenate([
        np.concatenate([np.arange(FS * s, FS * s + FS),
                        np.arange(F + FS * s, F + FS * s + FS),
                        np.arange(2 * F + FS * s, 2 * F + FS * s + FS)])
        for s in range(NSHARD)
    ])
    w2p = W2[perm]
    b2p = b2[perm]

    xp = _mlp_tc(q[:, 0, :], W1.T, b1, w2p.T, b2p)
    x4 = xp.reshape(n, NSHARD, 3 * FS)

    mu_r = mu.reshape(n, 3, NSHARD, FS).transpose(0, 2, 1, 3)
    mu4 = mu_r.reshape(n, NSHARD, 3 * FS)
    xmu4 = jnp.concatenate([x4, mu4], axis=-1).transpose(1, 0, 2)
    q_r = q.reshape(n, 1, NSHARD, FS).transpose(0, 2, 1, 3)
    init4 = jnp.concatenate([q_r, mu_r], axis=2)
    init4 = init4.transpose(1, 0, 2, 3).reshape(NSHARD, n, 4 * FS)

    q_out, mu_out = _edges_sc(xmu4, init4, Wij.reshape(e, 3 * F),
                              dir_ij.reshape(-1), idx_i, idx_j)
    return (q_out.astype(q.dtype), mu_out.astype(mu.dtype))


# speedup vs baseline: 1.8432x; 1.0367x over previous
import functools

import numpy as np
import jax
import jax.numpy as jnp
from jax import lax
from jax.experimental import pallas as pl
from jax.experimental.pallas import tpu as pltpu
from jax.experimental.pallas import tpu_sc as plsc

F = 128
FS = 32
NSHARD = 4
NTILES = 16
CCHUNK = 80


def _mlp_tc(q2, w1t, b1, w2pt, b2p):
    n = q2.shape[0]
    blk = 400

    def body(q_ref, w1t_ref, b1_ref, w2pt_ref, b2p_ref, out_ref):
        h = jnp.dot(q_ref[...], w1t_ref[...], preferred_element_type=jnp.float32)
        h = h + b1_ref[...]
        h = h * jax.nn.sigmoid(h)
        x = jnp.dot(h, w2pt_ref[...], preferred_element_type=jnp.float32)
        out_ref[...] = x + b2p_ref[...]

    return pl.pallas_call(
        body,
        grid=(n // blk,),
        in_specs=[
            pl.BlockSpec((blk, F), lambda i: (i, 0)),
            pl.BlockSpec((F, F), lambda i: (0, 0)),
            pl.BlockSpec((1, F), lambda i: (0, 0)),
            pl.BlockSpec((F, 3 * F), lambda i: (0, 0)),
            pl.BlockSpec((1, 3 * F), lambda i: (0, 0)),
        ],
        out_specs=pl.BlockSpec((blk, 3 * F), lambda i: (i, 0)),
        out_shape=jax.ShapeDtypeStruct((n, 3 * F), jnp.float32),
    )(q2, w1t, b1.reshape(1, F), w2pt, b2p.reshape(1, 3 * F))


def _edges_sc(xmu4, init4, wij, dirij, idx_i, idx_j):
    n = init4.shape[1]
    e = wij.shape[0]
    e_tile = e // NTILES
    n_tile = n // NTILES
    nchunk = e_tile // CCHUNK
    npair = (nchunk - 1) // 2
    mesh = plsc.VectorSubcoreMesh(core_axis_name="c", subcore_axis_name="s")
    vm = pltpu.VMEM

    @functools.partial(
        pl.kernel,
        mesh=mesh,
        compiler_params=pltpu.CompilerParams(use_tc_tiling_on_sc=False),
        out_type=[
            jax.ShapeDtypeStruct((n, 1, F), jnp.float32),
            jax.ShapeDtypeStruct((n, 3, F), jnp.float32),
        ],
        scratch_types=[
            pltpu.VMEM_SHARED((n, F), jnp.float32),
            [vm((2, CCHUNK // 2), jnp.int32)] * 2,
            [vm((CCHUNK,), jnp.int32)] * 2,
            [vm((CCHUNK * 3 + 16,), jnp.float32)] * 2,
            vm((CCHUNK, FS), jnp.float32),
            vm((CCHUNK, FS), jnp.float32),
            vm((CCHUNK, FS), jnp.float32),
            [vm((CCHUNK, 6 * FS), jnp.float32)] * 2,
            vm((CCHUNK // 2, 4 * FS), jnp.float32),
            vm((CCHUNK // 2, 4 * FS), jnp.float32),
            [pltpu.SemaphoreType.DMA] * 2,
            [pltpu.SemaphoreType.DMA] * 2,
            pltpu.SemaphoreType.DMA,
            pltpu.SemaphoreType.DMA,
        ],
    )
    def k(xmu4_h, init4_h, wij_h, dir_h, idxi_h, idxj_h, qout_h, muout_h,
          acc, idxi_v, idxj_v, dir_v, wd_v, wa_v, wb_v, xm_v, paya_v, payb_v,
          semj, semlg, semw, sems):
        c = lax.axis_index("c")
        s = lax.axis_index("s")
        rows0 = s * n_tile
        ebase = s * e_tile

        rowbase = s * (e_tile // (CCHUNK // 2))

        def issue_idx(kk, b):
            e0 = ebase + kk * CCHUNK
            pltpu.async_copy(idxj_h.at[pl.ds(e0, CCHUNK)], idxj_v[b], semj[b])
            pltpu.async_copy(idxi_h.at[pl.ds(rowbase + 2 * kk, 2)],
                             idxi_v[b], semlg[b])
            pltpu.async_copy(dir_h.at[pl.ds(3 * e0, 3 * CCHUNK)],
                             dir_v[b].at[pl.ds(0, 3 * CCHUNK)], semlg[b])

        def issue_wij(kk, shard):
            e0 = ebase + kk * CCHUNK
            for col, dst in ((0, wd_v), (F, wa_v), (2 * F, wb_v)):
                pltpu.async_copy(
                    wij_h.at[pl.ds(e0, CCHUNK), pl.ds(col + shard * FS, FS)],
                    dst, semw)

        def wait_j(b):
            pltpu.make_async_copy(idxj_h.at[pl.ds(0, CCHUNK)], idxj_v[b],
                                  semj[b]).wait()

        def wait_wij():
            for dst in (wd_v, wa_v, wb_v):
                pltpu.make_async_copy(
                    wij_h.at[pl.ds(0, CCHUNK), pl.ds(0, FS)], dst, semw).wait()

        def issue_gathers(b, shard):
            pltpu.async_copy(xmu4_h.at[shard].at[idxj_v[b]], xm_v[b], semlg[b])

        def wait_lg(b):
            pltpu.make_async_copy(idxi_h.at[pl.ds(0, 2)], idxi_v[b],
                                  semlg[b]).wait()
            pltpu.make_async_copy(dir_h.at[pl.ds(0, 3 * CCHUNK)],
                                  dir_v[b].at[pl.ds(0, 3 * CCHUNK)],
                                  semlg[b]).wait()
            pltpu.make_async_copy(xmu4_h.at[0].at[idxj_v[b]], xm_v[b],
                                  semlg[b]).wait()

        def compute_scatter(b):
            def emit_edge(ei, pay_ref, po):
                dvec = dir_v[b][pl.ds(3 * ei, 16)]
                d0 = dvec[0]
                d1 = dvec[1]
                d2 = dvec[2]
                for h in range(FS // 16):
                    hs = 16 * h
                    pay_ref[po, pl.ds(hs, 16)] = (
                        wd_v[ei, pl.ds(hs, 16)] * xm_v[b][ei, pl.ds(hs, 16)])
                    a = wa_v[ei, pl.ds(hs, 16)] * xm_v[b][ei, pl.ds(FS + hs, 16)]
                    bb = (wb_v[ei, pl.ds(hs, 16)]
                          * xm_v[b][ei, pl.ds(2 * FS + hs, 16)])
                    for d, dd in enumerate((d0, d1, d2)):
                        pay_ref[po, pl.ds(FS + FS * d + hs, 16)] = (
                            a * dd
                            + bb * xm_v[b][ei, pl.ds(3 * FS + FS * d + hs, 16)])

            half = CCHUNK // 2

            @plsc.parallel_loop(0, half, unroll=4)
            def edge_a(ei):
                emit_edge(ei, paya_v, ei)

            ha = pltpu.async_copy(paya_v, acc.at[idxi_v[b].at[0]], sems,
                                  add=True)

            @plsc.parallel_loop(half, CCHUNK, unroll=4)
            def edge_b(ei):
                emit_edge(ei, payb_v, ei - half)

            ha.wait()
            pltpu.sync_copy(payb_v, acc.at[idxi_v[b].at[1]], add=True)

        for p in range(2):
            shard = 2 * p + c

            pltpu.sync_copy(init4_h.at[shard, pl.ds(rows0, n_tile)],
                            acc.at[pl.ds(rows0, n_tile)])
            plsc.subcore_barrier()

            issue_idx(0, 0)
            issue_idx(1, 1)
            issue_wij(0, shard)
            wait_j(0)
            issue_gathers(0, shard)

            def pair_body(kp, _):
                for b in (0, 1):
                    kk = 2 * kp + b
                    wait_j(1 - b)
                    issue_gathers(1 - b, shard)
                    wait_wij()
                    wait_lg(b)
                    compute_scatter(b)
                    issue_wij(kk + 1, shard)

                    @pl.when(kk + 2 < nchunk)
                    def _():
                        issue_idx(kk + 2, b)
                return 0

            lax.fori_loop(0, npair, pair_body, 0)
            wait_wij()
            wait_lg(0)
            compute_scatter(0)
            plsc.subcore_barrier()

            col = shard * FS
            pltpu.sync_copy(acc.at[pl.ds(rows0, n_tile), pl.ds(0, FS)],
                            qout_h.at[pl.ds(rows0, n_tile), 0, pl.ds(col, FS)])
            for d in range(3):
                pltpu.sync_copy(
                    acc.at[pl.ds(rows0, n_tile), pl.ds(FS + FS * d, FS)],
                    muout_h.at[pl.ds(rows0, n_tile), d, pl.ds(col, FS)])
            if p == 0:
                plsc.subcore_barrier()

    return k(xmu4, init4, wij, dirij, idx_i, idx_j)


def kernel(q, mu, Wij, dir_ij, idx_i, idx_j, n_atoms, W1, b1, W2, b2):
    n = q.shape[0]
    e = Wij.shape[0]
    idx_i = idx_i.astype(jnp.int32)
    idx_j = idx_j.astype(jnp.int32)

    perm = np.concatenate([
        np.concatenate([np.arange(FS * s, FS * s + FS),
                        np.arange(F + FS * s, F + FS * s + FS),
                        np.arange(2 * F + FS * s, 2 * F + FS * s + FS)])
        for s in range(NSHARD)
    ])
    w2p = W2[perm]
    b2p = b2[perm]

    xp = _mlp_tc(q[:, 0, :], W1.T, b1, w2p.T, b2p)
    x4 = xp.reshape(n, NSHARD, 3 * FS)

    mu_r = mu.reshape(n, 3, NSHARD, FS).transpose(0, 2, 1, 3)
    mu4 = mu_r.reshape(n, NSHARD, 3 * FS)
    xmu4 = jnp.concatenate([x4, mu4], axis=-1).transpose(1, 0, 2)
    q_r = q.reshape(n, 1, NSHARD, FS).transpose(0, 2, 1, 3)
    init4 = jnp.concatenate([q_r, mu_r], axis=2)
    init4 = init4.transpose(1, 0, 2, 3).reshape(NSHARD, n, 4 * FS)

    q_out, mu_out = _edges_sc(xmu4, init4, Wij.reshape(e, 3 * F),
                              dir_ij.reshape(-1),
                              idx_i.reshape(-1, CCHUNK // 2), idx_j)
    return (q_out.astype(q.dtype), mu_out.astype(mu.dtype))
